# Initial kernel scaffold; baseline (speedup 1.0000x reference)
import jax
import jax.numpy as jnp
from jax.experimental import pallas as pl


def kernel(x, luts, mapping):
    raise NotImplementedError("write your pallas kernel here")


# speedup vs baseline: 3.9663x; 3.9663x over previous
import functools

import jax
import jax.numpy as jnp
from jax import lax
from jax.experimental import pallas as pl
from jax.experimental.pallas import tpu as pltpu
from jax.experimental.pallas import tpu_sc as plsc

BATCH = 1024
INPUT_SIZE = 1024
NUM_NODES = 1024
N_IN = 6
ROWS = N_IN * NUM_NODES

_NC, _NS = 2, 16
_NW = _NC * _NS
_ROWS_PER_W = ROWS // _NW
_CHUNK = 96


def _sc_gather(xt, flat_idx):
    @functools.partial(
        pl.kernel,
        mesh=plsc.VectorSubcoreMesh(core_axis_name="c", subcore_axis_name="s"),
        out_type=jax.ShapeDtypeStruct((ROWS, BATCH), jnp.float32),
        scratch_types=[
            pltpu.VMEM((_CHUNK,), jnp.int32),
            pltpu.VMEM((_CHUNK, BATCH), jnp.float32),
            pltpu.SemaphoreType.DMA,
        ],
    )
    def body(xt_hbm, idx_hbm, out_hbm, idx_v, rows_v, sem):
        wid = lax.axis_index("s") * _NC + lax.axis_index("c")
        base = wid * _ROWS_PER_W
        for c in range(_ROWS_PER_W // _CHUNK):
            off = base + c * _CHUNK
            pltpu.sync_copy(idx_hbm.at[pl.ds(off, _CHUNK)], idx_v)
            pltpu.async_copy(xt_hbm.at[idx_v], rows_v, sem).wait()
            pltpu.sync_copy(rows_v, out_hbm.at[pl.ds(off, _CHUNK)])

    return body(xt, flat_idx)


_NT = 8


def _fold_body(luts_ref, g_ref, out_ref):
    g = [g_ref[i] for i in range(N_IN)]

    def t(level, j):
        if level == 1:
            a = luts_ref[:, 2 * j:2 * j + 1]
            b = luts_ref[:, 2 * j + 1:2 * j + 2]
            return a + g[0] * (b - a)
        a = t(level - 1, 2 * j)
        b = t(level - 1, 2 * j + 1)
        return a + g[level - 1] * (b - a)

    out_ref[...] = t(N_IN, 0)


def _tc_fold(luts, g3):
    return pl.pallas_call(
        _fold_body,
        grid=(NUM_NODES // _NT,),
        in_specs=[
            pl.BlockSpec((_NT, 2 ** N_IN), lambda j: (j, 0)),
            pl.BlockSpec((N_IN, _NT, BATCH), lambda j: (0, j, 0)),
        ],
        out_specs=pl.BlockSpec((_NT, BATCH), lambda j: (j, 0)),
        out_shape=jax.ShapeDtypeStruct((NUM_NODES, BATCH), jnp.float32),
    )(luts, g3)


def kernel(x, luts, mapping):
    xt = x.T
    flat_idx = mapping.T.reshape(ROWS).astype(jnp.int32)
    g = _sc_gather(xt, flat_idx)
    g3 = g.reshape(N_IN, NUM_NODES, BATCH)
    out_t = _tc_fold(luts, g3)
    return out_t.T


# speedup vs baseline: 6.1484x; 1.5502x over previous
import functools

import jax
import jax.numpy as jnp
from jax import lax
from jax.experimental import pallas as pl
from jax.experimental.pallas import tpu as pltpu
from jax.experimental.pallas import tpu_sc as plsc

BATCH = 1024
INPUT_SIZE = 1024
NUM_NODES = 1024
N_IN = 6
ROWS = N_IN * NUM_NODES

_NC, _NS = 2, 16
_NW = _NC * _NS
_ROWS_PER_W = ROWS // _NW
_CHUNK = 96


def _sc_gather(xt, flat_idx):
    @functools.partial(
        pl.kernel,
        mesh=plsc.VectorSubcoreMesh(core_axis_name="c", subcore_axis_name="s"),
        out_type=jax.ShapeDtypeStruct((ROWS, BATCH), jnp.float32),
        scratch_types=[
            pltpu.VMEM((_CHUNK,), jnp.int32),
            pltpu.VMEM((_CHUNK, BATCH), jnp.float32),
            pltpu.SemaphoreType.DMA,
        ],
    )
    def body(xt_hbm, idx_hbm, out_hbm, idx_v, rows_v, sem):
        wid = lax.axis_index("s") * _NC + lax.axis_index("c")
        base = wid * _ROWS_PER_W
        for c in range(_ROWS_PER_W // _CHUNK):
            off = base + c * _CHUNK
            pltpu.sync_copy(idx_hbm.at[pl.ds(off, _CHUNK)], idx_v)
            pltpu.async_copy(xt_hbm.at[idx_v], rows_v, sem).wait()
            pltpu.sync_copy(rows_v, out_hbm.at[pl.ds(off, _CHUNK)])

    return body(xt, flat_idx)


_NT = 32


def _fold_body(luts_ref, g_ref, out_ref):
    g = [g_ref[i] for i in range(N_IN)]

    def t(level, j):
        if level == 1:
            a = luts_ref[:, 2 * j:2 * j + 1]
            b = luts_ref[:, 2 * j + 1:2 * j + 2]
            return a + g[0] * (b - a)
        a = t(level - 1, 2 * j)
        b = t(level - 1, 2 * j + 1)
        return a + g[level - 1] * (b - a)

    out_ref[...] = t(N_IN, 0)


def _tc_fold(luts, g3):
    return pl.pallas_call(
        _fold_body,
        grid=(NUM_NODES // _NT,),
        in_specs=[
            pl.BlockSpec((_NT, 2 ** N_IN), lambda j: (j, 0)),
            pl.BlockSpec((N_IN, _NT, BATCH), lambda j: (0, j, 0)),
        ],
        out_specs=pl.BlockSpec((_NT, BATCH), lambda j: (j, 0)),
        out_shape=jax.ShapeDtypeStruct((NUM_NODES, BATCH), jnp.float32),
    )(luts, g3)


def kernel(x, luts, mapping):
    xt = x.T
    flat_idx = mapping.T.reshape(ROWS).astype(jnp.int32)
    g = _sc_gather(xt, flat_idx)
    g3 = g.reshape(N_IN, NUM_NODES, BATCH)
    out_t = _tc_fold(luts, g3)
    return out_t.T


# speedup vs baseline: 6.7734x; 1.1016x over previous
import functools

import jax
import jax.numpy as jnp
from jax import lax
from jax.experimental import pallas as pl
from jax.experimental.pallas import tpu as pltpu
from jax.experimental.pallas import tpu_sc as plsc

BATCH = 1024
INPUT_SIZE = 1024
NUM_NODES = 1024
N_IN = 6
ROWS = N_IN * NUM_NODES

_NC, _NS = 2, 16
_NW = _NC * _NS
_ROWS_PER_W = ROWS // _NW
_CHUNK = 96


def _sc_gather(xt, flat_idx):
    @functools.partial(
        pl.kernel,
        mesh=plsc.VectorSubcoreMesh(core_axis_name="c", subcore_axis_name="s"),
        out_type=jax.ShapeDtypeStruct((ROWS, BATCH), jnp.float32),
        scratch_types=[
            pltpu.VMEM((_CHUNK,), jnp.int32),
            pltpu.VMEM((_CHUNK, BATCH), jnp.float32),
            pltpu.SemaphoreType.DMA,
        ],
    )
    def body(xt_hbm, idx_hbm, out_hbm, idx_v, rows_v, sem):
        wid = lax.axis_index("s") * _NC + lax.axis_index("c")
        base = wid * _ROWS_PER_W
        for c in range(_ROWS_PER_W // _CHUNK):
            off = base + c * _CHUNK
            pltpu.sync_copy(idx_hbm.at[pl.ds(off, _CHUNK)], idx_v)
            pltpu.async_copy(xt_hbm.at[idx_v], rows_v, sem).wait()
            pltpu.sync_copy(rows_v, out_hbm.at[pl.ds(off, _CHUNK)])

    return body(xt, flat_idx)


_NT = 128


def _fold_body(luts_ref, g_ref, out_ref):
    g = [g_ref[i] for i in range(N_IN)]

    def t(level, j):
        if level == 1:
            a = luts_ref[:, 2 * j:2 * j + 1]
            b = luts_ref[:, 2 * j + 1:2 * j + 2]
            return a + g[0] * (b - a)
        a = t(level - 1, 2 * j)
        b = t(level - 1, 2 * j + 1)
        return a + g[level - 1] * (b - a)

    out_ref[...] = t(N_IN, 0).T


def _tc_fold(luts, g3):
    return pl.pallas_call(
        _fold_body,
        grid=(NUM_NODES // _NT,),
        in_specs=[
            pl.BlockSpec((_NT, 2 ** N_IN), lambda j: (j, 0)),
            pl.BlockSpec((N_IN, _NT, BATCH), lambda j: (0, j, 0)),
        ],
        out_specs=pl.BlockSpec((BATCH, _NT), lambda j: (0, j)),
        out_shape=jax.ShapeDtypeStruct((BATCH, NUM_NODES), jnp.float32),
    )(luts, g3)


def kernel(x, luts, mapping):
    xt = x.T
    flat_idx = mapping.T.reshape(ROWS).astype(jnp.int32)
    g = _sc_gather(xt, flat_idx)
    g3 = g.reshape(N_IN, NUM_NODES, BATCH)
    return _tc_fold(luts, g3)


# speedup vs baseline: 7.4140x; 1.0946x over previous
import functools

import jax
import jax.numpy as jnp
from jax import lax
from jax.experimental import pallas as pl
from jax.experimental.pallas import tpu as pltpu
from jax.experimental.pallas import tpu_sc as plsc

BATCH = 1024
INPUT_SIZE = 1024
NUM_NODES = 1024
N_IN = 6

_NC, _NS = 2, 16
_NW = _NC * _NS

_HALF = NUM_NODES // 2
_HROWS = N_IN * _HALF
_CHUNK = _HROWS // _NW


def _sc_gather_half(xt, flat_idx):
    @functools.partial(
        pl.kernel,
        mesh=plsc.VectorSubcoreMesh(core_axis_name="c", subcore_axis_name="s"),
        out_type=jax.ShapeDtypeStruct((_HROWS, BATCH), jnp.float32),
        scratch_types=[
            pltpu.VMEM((_CHUNK,), jnp.int32),
            pltpu.VMEM((_CHUNK, BATCH), jnp.float32),
            pltpu.SemaphoreType.DMA,
        ],
    )
    def body(xt_hbm, idx_hbm, out_hbm, idx_v, rows_v, sem):
        wid = lax.axis_index("s") * _NC + lax.axis_index("c")
        off = wid * _CHUNK
        pltpu.sync_copy(idx_hbm.at[pl.ds(off, _CHUNK)], idx_v)
        pltpu.async_copy(xt_hbm.at[idx_v], rows_v, sem).wait()
        pltpu.sync_copy(rows_v, out_hbm.at[pl.ds(off, _CHUNK)])

    return body(xt, flat_idx)


_NT = 128


def _fold_body(luts_ref, g_ref, out_ref):
    g = [g_ref[i] for i in range(N_IN)]

    def t(level, j):
        if level == 1:
            a = luts_ref[:, 2 * j:2 * j + 1]
            b = luts_ref[:, 2 * j + 1:2 * j + 2]
            return a + g[0] * (b - a)
        a = t(level - 1, 2 * j)
        b = t(level - 1, 2 * j + 1)
        return a + g[level - 1] * (b - a)

    out_ref[...] = t(N_IN, 0).T


def _tc_fold_half(luts_h, g3, col0, prev=None):
    steps = _HALF // _NT
    base = col0 // _NT
    in_specs = [
        pl.BlockSpec((_NT, 2 ** N_IN), lambda j: (j, 0)),
        pl.BlockSpec((N_IN, _NT, BATCH), lambda j: (0, j, 0)),
    ]
    args = [luts_h, g3]
    aliases = {}
    body = _fold_body
    if prev is not None:
        in_specs.append(pl.BlockSpec(memory_space=pltpu.MemorySpace.HBM))
        args.append(prev)
        aliases = {2: 0}
        body = lambda l, g, _p, o: _fold_body(l, g, o)
    return pl.pallas_call(
        body,
        grid=(steps,),
        in_specs=in_specs,
        out_specs=pl.BlockSpec((BATCH, _NT), lambda j: (0, j + base)),
        out_shape=jax.ShapeDtypeStruct((BATCH, NUM_NODES), jnp.float32),
        input_output_aliases=aliases,
    )(*args)


def kernel(x, luts, mapping):
    xt = x.T
    m_t = mapping.T.astype(jnp.int32)
    idx_a = m_t[:, :_HALF].reshape(_HROWS)
    idx_b = m_t[:, _HALF:].reshape(_HROWS)
    g_a = _sc_gather_half(xt, idx_a).reshape(N_IN, _HALF, BATCH)
    g_b = _sc_gather_half(xt, idx_b).reshape(N_IN, _HALF, BATCH)
    out_a = _tc_fold_half(luts[:_HALF], g_a, 0)
    return _tc_fold_half(luts[_HALF:], g_b, _HALF, prev=out_a)
